# Initial kernel scaffold; baseline (speedup 1.0000x reference)
import jax
import jax.numpy as jnp
from jax.experimental import pallas as pl


def _dist_kernel(p_ref, t_ref, o_ref):
    p = p_ref[...]
    t = t_ref[...]
    pt = jax.lax.dot_general(
        p, t, (((1,), (1,)), ((), ())),
        preferred_element_type=jnp.float32)
    t_sq = jnp.sum(t * t, axis=1)
    d = t_sq[None, :] - 2.0 * pt
    m = jnp.min(d, axis=1)
    o_ref[0, 0] = jnp.sum(m) + jnp.sum(p * p)


def kernel(points, targets):
    out = pl.pallas_call(
        _dist_kernel,
        out_shape=jax.ShapeDtypeStruct((1, 1), jnp.float32),
    )(points, targets)
    return out[0, 0]


# speedup vs baseline: 32.9263x; 32.9263x over previous
import jax
import jax.numpy as jnp
from jax.experimental import pallas as pl


def _dist_kernel(p_ref, tt_ref, o_ref):
    p = p_ref[...]
    tt = tt_ref[...]
    pt = jnp.dot(p, tt, preferred_element_type=jnp.float32)
    t_sq = jnp.sum(tt * tt, axis=0)
    d = t_sq[None, :] - 2.0 * pt
    m = jnp.min(d, axis=1)
    total = jnp.sum(m) + jnp.sum(p * p)
    o_ref[...] = jnp.reshape(total, (1, 1))


def kernel(points, targets):
    out = pl.pallas_call(
        _dist_kernel,
        out_shape=jax.ShapeDtypeStruct((1, 1), jnp.float32),
    )(points, targets.T)
    return out[0, 0]
